# baseline (speedup 1.0000x reference)
import functools
import math

import jax
import jax.numpy as jnp
from jax import lax
from jax.experimental import pallas as pl
from jax.experimental.pallas import tpu as pltpu
from jax.experimental.pallas import tpu_sc as plsc
from jax._src.pallas import mpmd as _mpmd

H = 8
DK = 64
DM = 512
P = 64
NSEG = 128
SEG = 32
SEN = 4096
SEGS_PER_BLK = 32
NBLK = NSEG // SEGS_PER_BLK
ROWS_PER_BLK = SEGS_PER_BLK * SEG
NGATHER = NSEG + 8

@functools.cache
def _sc_gather_kernel():
    mesh = plsc.VectorSubcoreMesh(core_axis_name="c", subcore_axis_name="s")

    @functools.partial(
        pl.kernel,
        out_type=jax.ShapeDtypeStruct((NGATHER, DM), jnp.float32),
        mesh=mesh,
        scratch_types=[
            pltpu.VMEM((8,), jnp.int32),
            pltpu.VMEM((8, DM), jnp.float32),
            pltpu.SemaphoreType.DMA,
        ],
    )
    def _sc_gather(q_hbm, idx_hbm, out_hbm, idx_v, rows_v, sem):
        wid = lax.axis_index("s") * 2 + lax.axis_index("c")

        @pl.when(wid < NGATHER // 8)
        def _():
            base = wid * 8
            pltpu.sync_copy(idx_hbm.at[pl.ds(base, 8)], idx_v)
            pltpu.async_copy(q_hbm.at[idx_v], rows_v, sem).wait()
            pltpu.sync_copy(rows_v, out_hbm.at[pl.ds(base, 8)])

    return _sc_gather


@functools.cache
def _sc_scatter_kernel():
    mesh = plsc.VectorSubcoreMesh(core_axis_name="c", subcore_axis_name="s")

    def _sc_scatter(base_hbm, rows_hbm, idx_hbm, out_hbm, idx_v, rows_v, sem):
        del base_hbm
        wid = lax.axis_index("s") * 2 + lax.axis_index("c")

        @pl.when(wid < NSEG // 8)
        def _():
            base = wid * 8
            pltpu.sync_copy(idx_hbm.at[pl.ds(base, 8)], idx_v)
            pltpu.sync_copy(rows_hbm.at[pl.ds(base, 8)], rows_v)
            pltpu.async_copy(rows_v, out_hbm.at[idx_v], sem).wait()

    return _mpmd._mpmd_map(
        [(mesh, _sc_scatter)],
        out_types=jax.ShapeDtypeStruct((SEN, DM), jnp.float32),
        input_output_aliases={0: 0},
        scratch_types=[
            pltpu.VMEM((8,), jnp.int32),
            pltpu.VMEM((8, DM), jnp.float32),
            pltpu.SemaphoreType.DMA,
        ],
    )


def _path_attn(qp_ref, path_ref, wpk_ref, wpv_ref, wqp_ref, wo_ref, bout_ref,
               wagg_ref, att_ref, pr_ref):
    f32 = jnp.float32
    pk = jnp.dot(path_ref[...], wpk_ref[...], preferred_element_type=f32)
    pv = jnp.dot(path_ref[...], wpv_ref[...], preferred_element_type=f32)
    pq = (jnp.dot(qp_ref[0:1, :], wqp_ref[0:DM, :], preferred_element_type=f32)
          + jnp.dot(qp_ref[1:2, :], wqp_ref[DM:2 * DM, :],
                    preferred_element_type=f32))
    selr = lax.broadcasted_iota(jnp.int32, (DM, H), 0) // DK
    selc = lax.broadcasted_iota(jnp.int32, (DM, H), 1)
    sel = (selr == selc).astype(f32)
    selr_t = lax.broadcasted_iota(jnp.int32, (H, DM), 1) // DK
    selc_t = lax.broadcasted_iota(jnp.int32, (H, DM), 0)
    sel_t = (selr_t == selc_t).astype(f32)
    lg = jnp.dot(pk * pq, sel, preferred_element_type=f32) * (1.0 / math.sqrt(DK))
    m = jnp.max(lg, axis=0, keepdims=True)
    e = jnp.exp(lg - m)
    att_ph = e / jnp.sum(e, axis=0, keepdims=True)
    r64 = lax.broadcasted_iota(jnp.int32, (P, P), 0)
    c64 = lax.broadcasted_iota(jnp.int32, (P, P), 1)
    eye = (r64 == c64).astype(f32)
    att_ref[...] = lax.dot_general(att_ph, eye, (((0,), (0,)), ((), ())),
                                   preferred_element_type=f32)
    att_x = jnp.dot(att_ph, sel_t, preferred_element_type=f32)
    vals = jnp.sum(pv * att_x, axis=0, keepdims=True)
    path_res = (jnp.dot(vals, wo_ref[...],
                        preferred_element_type=f32) + bout_ref[...])
    pr_a = jnp.dot(path_res, wagg_ref[0:DM, :], preferred_element_type=f32)
    pr_b = jnp.dot(path_res, wagg_ref[DM:2 * DM, :], preferred_element_type=f32)
    pr_ref[0:1, :] = pr_a
    pr_ref[1:2, :] = pr_b


def _main_body(qg_ref, k_ref, v_ref, q_ref, qp_ref, path_ref, tr_ref, tc_ref,
               wq_ref, wk_ref, wv_ref, wo_ref, wagg_ref, wpk_ref, wpv_ref,
               wqp_ref, bout_ref, st_ref, out_ref, att_ref, base_ref,
               acc_ref, pr_ref, cp_sem):
    f32 = jnp.float32
    g = pl.program_id(0)
    bf16 = jnp.bfloat16

    rows_q = SEN // NBLK
    cp = pltpu.make_async_copy(q_ref.at[pl.ds(g * rows_q, rows_q), :],
                               base_ref.at[pl.ds(g * rows_q, rows_q), :],
                               cp_sem)
    cp.start()

    @pl.when(g == 0)
    def _():
        _path_attn(qp_ref, path_ref, wpk_ref, wpv_ref, wqp_ref, wo_ref,
                   bout_ref, wagg_ref, att_ref, pr_ref)

    qv = jnp.dot(qg_ref[...], wq_ref[...],
                 preferred_element_type=f32).astype(bf16)
    kp = jnp.dot(k_ref[...].astype(bf16), wk_ref[...].astype(bf16),
                 preferred_element_type=f32).astype(bf16)
    vp = jnp.dot(v_ref[...].astype(bf16), wv_ref[...].astype(bf16),
                 preferred_element_type=f32)

    prA = pr_ref[0:1, :]
    prB = pr_ref[1:2, :]
    src = st_ref[0]
    tar = st_ref[1]
    base = g * SEGS_PER_BLK

    selr = lax.broadcasted_iota(jnp.int32, (DM, H), 0) // DK
    selc = lax.broadcasted_iota(jnp.int32, (DM, H), 1)
    S = (selr == selc).astype(bf16)
    t8r = lax.broadcasted_iota(jnp.int32, (H, DM), 1) // DK
    t8c = lax.broadcasted_iota(jnp.int32, (H, DM), 0)
    T8 = (t8r == t8c).astype(f32)
    mr = lax.broadcasted_iota(jnp.int32, (H * H, H), 0)
    mb = lax.broadcasted_iota(jnp.int32, (H * H, H), 1)
    MM = (mr % H == mb).astype(f32)

    lg_cols = []
    for o in range(H):
        qr = qv if o == 0 else jnp.concatenate(
            [qv[:, o * DK:], qv[:, :o * DK]], axis=1)
        qE = jnp.broadcast_to(
            qr[:, None, :], (SEGS_PER_BLK, SEG, DM)).reshape(ROWS_PER_BLK, DM)
        lg_cols.append(jnp.dot(kp * qE, S, preferred_element_type=f32))
    LB = jnp.concatenate(lg_cols, axis=1)
    LB3 = LB.reshape(SEGS_PER_BLK, SEG, H * H) * (1.0 / math.sqrt(DK))
    m = jnp.max(LB3, axis=1, keepdims=True)
    e = jnp.exp(LB3 - m)
    att = e / jnp.sum(e, axis=1, keepdims=True)
    w2 = jnp.dot(att.reshape(ROWS_PER_BLK, H * H), MM,
                 preferred_element_type=f32)
    wX = jnp.dot(w2, T8, preferred_element_type=f32)
    fr_blk = jnp.sum((vp * wX).reshape(SEGS_PER_BLK, SEG, DM), axis=1)
    feature = (jnp.dot(fr_blk, wo_ref[...], preferred_element_type=f32)
               + bout_ref[...])
    fw_top = jnp.dot(feature, wagg_ref[0:DM, :], preferred_element_type=f32)
    fw_bot = jnp.dot(feature, wagg_ref[DM:2 * DM, :], preferred_element_type=f32)

    tt = tc_ref[pl.ds(base, SEGS_PER_BLK), :]
    m_src = (tt == src).astype(f32)
    m_tar = jnp.logical_and(tt == tar, tt != src).astype(f32)
    rows_blk = ((1.0 - m_src - m_tar) * feature
                + m_src * (fw_top + prB) + m_tar * (fw_bot + prA))
    acc_ref[pl.ds(base, SEGS_PER_BLK), :] = rows_blk
    cp.wait()

    @pl.when(g == NBLK - 1)
    def _():
        a = jnp.broadcast_to(tr_ref[...], (NSEG, NSEG))
        b = jnp.broadcast_to(tc_ref[...], (NSEG, NSEG))
        jidx = lax.broadcasted_iota(jnp.int32, (NSEG, NSEG), 1)
        last = jnp.max(jnp.where(a == b, jidx, -1), axis=1, keepdims=True)
        onehot = (jidx == last).astype(f32)
        out_ref[...] = jnp.dot(onehot, acc_ref[...], preferred_element_type=f32)


def _main_call(qg, kmat, vmat, q, qp, path, tid_row, tid_col, wq, wk, wv,
               wout, wagg, wpk, wpv, wqp, bout2, st2):
    const = lambda shape: pl.BlockSpec(shape, lambda g: (0, 0))
    return pl.pallas_call(
        _main_body,
        grid=(NBLK,),
        in_specs=[
            pl.BlockSpec((SEGS_PER_BLK, DM), lambda g: (g, 0)),
            pl.BlockSpec((ROWS_PER_BLK, DM), lambda g: (g, 0)),
            pl.BlockSpec((ROWS_PER_BLK, DM), lambda g: (g, 0)),
            pl.BlockSpec(memory_space=pltpu.MemorySpace.HBM),
            const((2, DM)),
            const((P, DM)),
            const((1, NSEG)),
            const((NSEG, 1)),
            const((DM, DM)),
            const((DM, DM)),
            const((DM, DM)),
            const((DM, DM)),
            const((2 * DM, DM)),
            const((DM, DM)),
            const((DM, DM)),
            const((2 * DM, DM)),
            const((1, DM)),
            pl.BlockSpec(memory_space=pltpu.SMEM),
        ],
        out_specs=(pl.BlockSpec((NSEG, DM), lambda g: (0, 0)),
                   pl.BlockSpec((H, P), lambda g: (0, 0)),
                   pl.BlockSpec(memory_space=pltpu.MemorySpace.HBM)),
        out_shape=(jax.ShapeDtypeStruct((NSEG, DM), jnp.float32),
                   jax.ShapeDtypeStruct((H, P), jnp.float32),
                   jax.ShapeDtypeStruct((SEN, DM), jnp.float32)),
        scratch_shapes=[pltpu.VMEM((NSEG, DM), jnp.float32),
                        pltpu.VMEM((8, DM), jnp.float32),
                        pltpu.SemaphoreType.DMA],
    )(qg, kmat, vmat, q, qp, path, tid_row, tid_col, wq, wk, wv, wout, wagg,
      wpk, wpv, wqp, bout2, st2)


def kernel(path, path_len, q, k, v, graphs, edge_len, token_ids, pair, rev,
           Wq_pair, Wpk, Wpv, Wq, Wk, Wv, Wout, bout, Wagg):
    src = jnp.where(rev == 0, pair[0], pair[1]).astype(jnp.int32)
    tar = jnp.where(rev == 0, pair[1], pair[0]).astype(jnp.int32)
    idx = jnp.concatenate([token_ids.astype(jnp.int32),
                           src[None], tar[None],
                           jnp.zeros((NGATHER - NSEG - 2,), jnp.int32)])

    qg = _sc_gather_kernel()(q, idx)

    bout2 = bout.reshape(1, DM)
    tid32 = token_ids.astype(jnp.int32)
    rows, att8, base = _main_call(qg[:NSEG], k, v, q, qg[NSEG:NSEG + 2], path,
                                  tid32.reshape(1, NSEG),
                                  tid32.reshape(NSEG, 1),
                                  Wq, Wk, Wv, Wout, Wagg, Wpk, Wpv, Wq_pair,
                                  bout2, jnp.stack([src, tar]))

    returned = _sc_scatter_kernel()(base, rows, tid32)
    return (returned, att8.reshape(H, 1, P))


# speedup vs baseline: 4.8047x; 4.8047x over previous
import functools
import math

import jax
import jax.numpy as jnp
from jax import lax
from jax.experimental import pallas as pl
from jax.experimental.pallas import tpu as pltpu
from jax.experimental.pallas import tpu_sc as plsc
from jax._src.pallas import mpmd as _mpmd

H = 8
DK = 64
DM = 512
P = 64
NSEG = 128
SEG = 32
SEN = 4096
SEGS_PER_BLK = 16
NBLK = NSEG // SEGS_PER_BLK
ROWS_PER_BLK = SEGS_PER_BLK * SEG
NGATHER = NSEG + 8

@functools.cache
def _sc_gather_kernel():
    mesh = plsc.VectorSubcoreMesh(core_axis_name="c", subcore_axis_name="s")

    @functools.partial(
        pl.kernel,
        out_type=jax.ShapeDtypeStruct((NGATHER, DM), jnp.float32),
        mesh=mesh,
        scratch_types=[
            pltpu.VMEM((8,), jnp.int32),
            pltpu.VMEM((8, DM), jnp.float32),
            pltpu.SemaphoreType.DMA,
        ],
    )
    def _sc_gather(q_hbm, idx_hbm, out_hbm, idx_v, rows_v, sem):
        wid = lax.axis_index("s") * 2 + lax.axis_index("c")

        @pl.when(wid < NGATHER // 8)
        def _():
            base = wid * 8
            pltpu.sync_copy(idx_hbm.at[pl.ds(base, 8)], idx_v)
            pltpu.async_copy(q_hbm.at[idx_v], rows_v, sem).wait()
            pltpu.sync_copy(rows_v, out_hbm.at[pl.ds(base, 8)])

    return _sc_gather


@functools.cache
def _sc_scatter_kernel():
    mesh = plsc.VectorSubcoreMesh(core_axis_name="c", subcore_axis_name="s")

    def _sc_scatter(base_hbm, rows_hbm, idx_hbm, out_hbm, idx_v, rows_v, sem):
        del base_hbm
        wid = lax.axis_index("s") * 2 + lax.axis_index("c")

        @pl.when(wid < NSEG // 8)
        def _():
            base = wid * 8
            pltpu.sync_copy(idx_hbm.at[pl.ds(base, 8)], idx_v)
            pltpu.sync_copy(rows_hbm.at[pl.ds(base, 8)], rows_v)
            pltpu.async_copy(rows_v, out_hbm.at[idx_v], sem).wait()

    return _mpmd._mpmd_map(
        [(mesh, _sc_scatter)],
        out_types=jax.ShapeDtypeStruct((SEN, DM), jnp.float32),
        input_output_aliases={0: 0},
        scratch_types=[
            pltpu.VMEM((8,), jnp.int32),
            pltpu.VMEM((8, DM), jnp.float32),
            pltpu.SemaphoreType.DMA,
        ],
    )


def _path_attn(qp_ref, path_ref, wpk_ref, wpv_ref, wqp_ref, wo_ref, bout_ref,
               wagg_ref, att_ref, pr_ref):
    f32 = jnp.float32
    pk = jnp.dot(path_ref[...], wpk_ref[...], preferred_element_type=f32)
    pv = jnp.dot(path_ref[...], wpv_ref[...], preferred_element_type=f32)
    pq = (jnp.dot(qp_ref[0:1, :], wqp_ref[0:DM, :], preferred_element_type=f32)
          + jnp.dot(qp_ref[1:2, :], wqp_ref[DM:2 * DM, :],
                    preferred_element_type=f32))
    selr = lax.broadcasted_iota(jnp.int32, (DM, H), 0) // DK
    selc = lax.broadcasted_iota(jnp.int32, (DM, H), 1)
    sel = (selr == selc).astype(f32)
    selr_t = lax.broadcasted_iota(jnp.int32, (H, DM), 1) // DK
    selc_t = lax.broadcasted_iota(jnp.int32, (H, DM), 0)
    sel_t = (selr_t == selc_t).astype(f32)
    lg = jnp.dot(pk * pq, sel, preferred_element_type=f32) * (1.0 / math.sqrt(DK))
    m = jnp.max(lg, axis=0, keepdims=True)
    e = jnp.exp(lg - m)
    att_ph = e / jnp.sum(e, axis=0, keepdims=True)
    r64 = lax.broadcasted_iota(jnp.int32, (P, P), 0)
    c64 = lax.broadcasted_iota(jnp.int32, (P, P), 1)
    eye = (r64 == c64).astype(f32)
    att_ref[...] = lax.dot_general(att_ph, eye, (((0,), (0,)), ((), ())),
                                   preferred_element_type=f32)
    att_x = jnp.dot(att_ph, sel_t, preferred_element_type=f32)
    vals = jnp.sum(pv * att_x, axis=0, keepdims=True)
    path_res = (jnp.dot(vals, wo_ref[...],
                        preferred_element_type=f32) + bout_ref[...])
    pr_a = jnp.dot(path_res, wagg_ref[0:DM, :], preferred_element_type=f32)
    pr_b = jnp.dot(path_res, wagg_ref[DM:2 * DM, :], preferred_element_type=f32)
    pr_ref[0:1, :] = pr_a
    pr_ref[1:2, :] = pr_b


def _main_body(qg_ref, k_ref, v_ref, q_ref, qp_ref, path_ref, tr_ref, tc_ref,
               wq_ref, wk_ref, wv_ref, wo_ref, wagg_ref, wpk_ref, wpv_ref,
               wqp_ref, bout_ref, st_ref, out_ref, att_ref, base_ref,
               acc_ref, pr_ref):
    f32 = jnp.float32
    g = pl.program_id(0)
    bf16 = jnp.bfloat16

    base_ref[...] = q_ref[...]

    @pl.when(g == 0)
    def _():
        _path_attn(qp_ref, path_ref, wpk_ref, wpv_ref, wqp_ref, wo_ref,
                   bout_ref, wagg_ref, att_ref, pr_ref)

    qv = jnp.dot(qg_ref[...], wq_ref[...],
                 preferred_element_type=f32).astype(bf16)
    kp = jnp.dot(k_ref[...].astype(bf16), wk_ref[...].astype(bf16),
                 preferred_element_type=f32).astype(bf16)
    vp = jnp.dot(v_ref[...].astype(bf16), wv_ref[...].astype(bf16),
                 preferred_element_type=f32)

    prA = pr_ref[0:1, :]
    prB = pr_ref[1:2, :]
    src = st_ref[0]
    tar = st_ref[1]
    base = g * SEGS_PER_BLK

    selr = lax.broadcasted_iota(jnp.int32, (DM, H), 0) // DK
    selc = lax.broadcasted_iota(jnp.int32, (DM, H), 1)
    S = (selr == selc).astype(bf16)
    t8r = lax.broadcasted_iota(jnp.int32, (H, DM), 1) // DK
    t8c = lax.broadcasted_iota(jnp.int32, (H, DM), 0)
    T8 = (t8r == t8c).astype(f32)
    mr = lax.broadcasted_iota(jnp.int32, (H * H, H), 0)
    mb = lax.broadcasted_iota(jnp.int32, (H * H, H), 1)
    MM = (mr % H == mb).astype(f32)

    lg_cols = []
    for o in range(H):
        qr = qv if o == 0 else jnp.concatenate(
            [qv[:, o * DK:], qv[:, :o * DK]], axis=1)
        qE = jnp.broadcast_to(
            qr[:, None, :], (SEGS_PER_BLK, SEG, DM)).reshape(ROWS_PER_BLK, DM)
        lg_cols.append(jnp.dot(kp * qE, S, preferred_element_type=f32))
    LB = jnp.concatenate(lg_cols, axis=1)
    LB3 = LB.reshape(SEGS_PER_BLK, SEG, H * H) * (1.0 / math.sqrt(DK))
    m = jnp.max(LB3, axis=1, keepdims=True)
    e = jnp.exp(LB3 - m)
    att = e / jnp.sum(e, axis=1, keepdims=True)
    w2 = jnp.dot(att.reshape(ROWS_PER_BLK, H * H), MM,
                 preferred_element_type=f32)
    wX = jnp.dot(w2, T8, preferred_element_type=f32)
    fr_blk = jnp.sum((vp * wX).reshape(SEGS_PER_BLK, SEG, DM), axis=1)
    feature = (jnp.dot(fr_blk, wo_ref[...], preferred_element_type=f32)
               + bout_ref[...])
    fw_top = jnp.dot(feature, wagg_ref[0:DM, :], preferred_element_type=f32)
    fw_bot = jnp.dot(feature, wagg_ref[DM:2 * DM, :], preferred_element_type=f32)

    tt = tc_ref[pl.ds(base, SEGS_PER_BLK), :]
    m_src = (tt == src).astype(f32)
    m_tar = jnp.logical_and(tt == tar, tt != src).astype(f32)
    rows_blk = ((1.0 - m_src - m_tar) * feature
                + m_src * (fw_top + prB) + m_tar * (fw_bot + prA))
    acc_ref[pl.ds(base, SEGS_PER_BLK), :] = rows_blk

    @pl.when(g == NBLK - 1)
    def _():
        a = jnp.broadcast_to(tr_ref[...], (NSEG, NSEG))
        b = jnp.broadcast_to(tc_ref[...], (NSEG, NSEG))
        jidx = lax.broadcasted_iota(jnp.int32, (NSEG, NSEG), 1)
        last = jnp.max(jnp.where(a == b, jidx, -1), axis=1, keepdims=True)
        onehot = (jidx == last).astype(f32)
        out_ref[...] = jnp.dot(onehot, acc_ref[...], preferred_element_type=f32)


def _main_call(qg, kmat, vmat, q, qp, path, tid_row, tid_col, wq, wk, wv,
               wout, wagg, wpk, wpv, wqp, bout2, st2):
    const = lambda shape: pl.BlockSpec(shape, lambda g: (0, 0))
    return pl.pallas_call(
        _main_body,
        grid=(NBLK,),
        in_specs=[
            pl.BlockSpec((SEGS_PER_BLK, DM), lambda g: (g, 0)),
            pl.BlockSpec((ROWS_PER_BLK, DM), lambda g: (g, 0)),
            pl.BlockSpec((ROWS_PER_BLK, DM), lambda g: (g, 0)),
            pl.BlockSpec((SEN // NBLK, DM), lambda g: (g, 0)),
            const((2, DM)),
            const((P, DM)),
            const((1, NSEG)),
            const((NSEG, 1)),
            const((DM, DM)),
            const((DM, DM)),
            const((DM, DM)),
            const((DM, DM)),
            const((2 * DM, DM)),
            const((DM, DM)),
            const((DM, DM)),
            const((2 * DM, DM)),
            const((1, DM)),
            pl.BlockSpec(memory_space=pltpu.SMEM),
        ],
        out_specs=(pl.BlockSpec((NSEG, DM), lambda g: (0, 0)),
                   pl.BlockSpec((H, P), lambda g: (0, 0)),
                   pl.BlockSpec((SEN // NBLK, DM), lambda g: (g, 0))),
        out_shape=(jax.ShapeDtypeStruct((NSEG, DM), jnp.float32),
                   jax.ShapeDtypeStruct((H, P), jnp.float32),
                   jax.ShapeDtypeStruct((SEN, DM), jnp.float32)),
        scratch_shapes=[pltpu.VMEM((NSEG, DM), jnp.float32),
                        pltpu.VMEM((8, DM), jnp.float32)],
    )(qg, kmat, vmat, q, qp, path, tid_row, tid_col, wq, wk, wv, wout, wagg,
      wpk, wpv, wqp, bout2, st2)


def kernel(path, path_len, q, k, v, graphs, edge_len, token_ids, pair, rev,
           Wq_pair, Wpk, Wpv, Wq, Wk, Wv, Wout, bout, Wagg):
    src = jnp.where(rev == 0, pair[0], pair[1]).astype(jnp.int32)
    tar = jnp.where(rev == 0, pair[1], pair[0]).astype(jnp.int32)
    idx = jnp.concatenate([token_ids.astype(jnp.int32),
                           src[None], tar[None],
                           jnp.zeros((NGATHER - NSEG - 2,), jnp.int32)])

    qg = _sc_gather_kernel()(q, idx)

    bout2 = bout.reshape(1, DM)
    tid32 = token_ids.astype(jnp.int32)
    rows, att8, base = _main_call(qg[:NSEG], k, v, q, qg[NSEG:NSEG + 2], path,
                                  tid32.reshape(1, NSEG),
                                  tid32.reshape(NSEG, 1),
                                  Wq, Wk, Wv, Wout, Wagg, Wpk, Wpv, Wq_pair,
                                  bout2, jnp.stack([src, tar]))

    returned = _sc_scatter_kernel()(base, rows, tid32)
    return (returned, att8.reshape(H, 1, P))
